# baseline (speedup 1.0000x reference)
import functools

import jax
import jax.numpy as jnp
from jax import lax
from jax.experimental import pallas as pl
from jax.experimental.pallas import tpu as pltpu
from jax.experimental.pallas import tpu_sc as plsc

L = 16


@functools.lru_cache(maxsize=None)
def _make_sc_partials(B, K, D, V):
    info = plsc.get_sparse_core_info()
    NC, NS = info.num_cores, info.num_subcores
    NW = NC * NS
    assert B % NW == 0
    BPW = B // NW
    BC = 32
    assert BPW % BC == 0
    NCHUNK = BPW // BC
    DV = D // L

    mesh = plsc.VectorSubcoreMesh(core_axis_name="c", subcore_axis_name="s")

    @functools.partial(
        pl.kernel,
        mesh=mesh,
        compiler_params=pltpu.CompilerParams(use_tc_tiling_on_sc=False),
        out_type=[
            jax.ShapeDtypeStruct((B * L,), jnp.float32),
            jax.ShapeDtypeStruct((B * K * L,), jnp.float32),
        ],
        scratch_types=[
            pltpu.VMEM((BC,), jnp.int32),
            pltpu.VMEM((BC,), jnp.int32),
            pltpu.VMEM((BC * K,), jnp.int32),
            pltpu.VMEM((BC, D), jnp.float32),
            pltpu.VMEM((BC, D), jnp.float32),
            pltpu.VMEM((BC * K, D), jnp.float32),
            pltpu.VMEM((BC * L,), jnp.float32),
            pltpu.VMEM((BC * K * L,), jnp.float32),
            pltpu.SemaphoreType.DMA,
        ],
    )
    def sc_partials(center_h, context_h, negflat_h, cemb_h, oemb_h,
                    pos_h, negs_h,
                    c_idx, o_idx, n_idx, c_rows, o_rows, n_rows,
                    pos_p, neg_p, sem):
        wid = lax.axis_index("s") * NC + lax.axis_index("c")
        base = wid * BPW

        def chunk_body(g, carry):
            b0 = pl.multiple_of(base + g * BC, BC)
            pltpu.sync_copy(center_h.at[pl.ds(b0, BC)], c_idx)
            pltpu.sync_copy(context_h.at[pl.ds(b0, BC)], o_idx)
            pltpu.sync_copy(negflat_h.at[pl.ds(b0 * K, BC * K)], n_idx)
            cp1 = pltpu.async_copy(cemb_h.at[c_idx], c_rows, sem)
            cp2 = pltpu.async_copy(oemb_h.at[o_idx], o_rows, sem)
            cp3 = pltpu.async_copy(oemb_h.at[n_idx], n_rows, sem)
            cp1.wait()
            cp2.wait()
            cp3.wait()

            def b_body(i, carry2):
                c = [c_rows[i, pl.ds(L * j, L)] for j in range(DV)]
                o = [o_rows[i, pl.ds(L * j, L)] for j in range(DV)]
                p = c[0] * o[0]
                for j in range(1, DV):
                    p = p + c[j] * o[j]
                pos_p[pl.ds(i * L, L)] = p
                for k in range(K):
                    r = i * K + k
                    q = c[0] * n_rows[r, pl.ds(0, L)]
                    for j in range(1, DV):
                        q = q + c[j] * n_rows[r, pl.ds(L * j, L)]
                    neg_p[pl.ds(r * L, L)] = q
                return carry2

            lax.fori_loop(0, BC, b_body, 0)
            pltpu.sync_copy(pos_p, pos_h.at[pl.ds(b0 * L, BC * L)])
            pltpu.sync_copy(neg_p, negs_h.at[pl.ds(b0 * K * L, BC * K * L)])
            return carry

        lax.fori_loop(0, NCHUNK, chunk_body, 0)

    return sc_partials


def _loss_body(pos_ref, neg_ref, out_ref, *, inv_b):
    red = (jax.lax.broadcasted_iota(jnp.int32, (128, 8), 0) // L
           == jax.lax.broadcasted_iota(jnp.int32, (128, 8), 1)
           ).astype(jnp.float32)

    def log_sigmoid(x):
        return jnp.minimum(x, 0.0) - jnp.log1p(jnp.exp(-jnp.abs(x)))

    pos = jax.lax.dot(pos_ref[...], red,
                      preferred_element_type=jnp.float32)
    neg = jax.lax.dot(neg_ref[...], red,
                      preferred_element_type=jnp.float32)
    total = jnp.sum(log_sigmoid(pos)) + jnp.sum(log_sigmoid(-neg))
    out_ref[...] = jnp.full((1, 1), -total * inv_b, dtype=jnp.float32)


def kernel(center, context, negative, center_emb, context_emb):
    B, K = negative.shape
    V, D = center_emb.shape
    sc_partials = _make_sc_partials(B, K, D, V)
    pos_p, neg_p = sc_partials(
        center.astype(jnp.int32),
        context.astype(jnp.int32),
        negative.reshape(B * K).astype(jnp.int32),
        center_emb,
        context_emb,
    )
    loss = pl.pallas_call(
        functools.partial(_loss_body, inv_b=1.0 / B),
        out_shape=jax.ShapeDtypeStruct((1, 1), jnp.float32),
    )(pos_p.reshape(B * L // 128, 128), neg_p.reshape(B * K * L // 128, 128))
    return loss[0, 0]


# speedup vs baseline: 1.0372x; 1.0372x over previous
import functools

import jax
import jax.numpy as jnp
from jax import lax
from jax.experimental import pallas as pl
from jax.experimental.pallas import tpu as pltpu
from jax.experimental.pallas import tpu_sc as plsc

L = 16


@functools.lru_cache(maxsize=None)
def _make_sc_partials(B, K, D, V):
    info = plsc.get_sparse_core_info()
    NC, NS = info.num_cores, info.num_subcores
    NW = NC * NS
    assert B % NW == 0
    BPW = B // NW
    BC = 32
    assert BPW % BC == 0
    NCHUNK = BPW // BC
    DV = D // L

    mesh = plsc.VectorSubcoreMesh(core_axis_name="c", subcore_axis_name="s")

    row_buf = lambda n: pltpu.VMEM((n, D), jnp.float32)

    @functools.partial(
        pl.kernel,
        mesh=mesh,
        compiler_params=pltpu.CompilerParams(use_tc_tiling_on_sc=False),
        out_type=[
            jax.ShapeDtypeStruct((B * L,), jnp.float32),
            jax.ShapeDtypeStruct((B * K * L,), jnp.float32),
        ],
        scratch_types=[
            pltpu.VMEM((BPW,), jnp.int32),
            pltpu.VMEM((BPW,), jnp.int32),
            pltpu.VMEM((BPW * K,), jnp.int32),
            [row_buf(BC), row_buf(BC), row_buf(BC * K)],
            [row_buf(BC), row_buf(BC), row_buf(BC * K)],
            pltpu.VMEM((BC * L,), jnp.float32),
            pltpu.VMEM((BC * K * L,), jnp.float32),
            pltpu.SemaphoreType.DMA,
            pltpu.SemaphoreType.DMA,
        ],
    )
    def sc_partials(center_h, context_h, negflat_h, cemb_h, oemb_h,
                    pos_h, negs_h,
                    c_idx, o_idx, n_idx, bufs0, bufs1,
                    pos_p, neg_p, sem0, sem1):
        wid = lax.axis_index("s") * NC + lax.axis_index("c")
        base = pl.multiple_of(wid * BPW, BPW)
        pltpu.sync_copy(center_h.at[pl.ds(base, BPW)], c_idx)
        pltpu.sync_copy(context_h.at[pl.ds(base, BPW)], o_idx)
        pltpu.sync_copy(negflat_h.at[pl.ds(base * K, BPW * K)], n_idx)

        bufs = (bufs0, bufs1)
        sems = (sem0, sem1)

        def fire(g, par):
            c_rows, o_rows, n_rows = bufs[par]
            sem = sems[par]
            return (
                pltpu.async_copy(
                    cemb_h.at[c_idx.at[pl.ds(g * BC, BC)]], c_rows, sem),
                pltpu.async_copy(
                    oemb_h.at[o_idx.at[pl.ds(g * BC, BC)]], o_rows, sem),
                pltpu.async_copy(
                    oemb_h.at[n_idx.at[pl.ds(g * BC * K, BC * K)]], n_rows,
                    sem),
            )

        descs = {par: fire(par, par) for par in range(2)}
        for g in range(NCHUNK):
            par = g % 2
            for d in descs[par]:
                d.wait()
            c_rows, o_rows, n_rows = bufs[par]

            def b_body(i, carry, c_rows=c_rows, o_rows=o_rows,
                       n_rows=n_rows):
                c = [c_rows[i, pl.ds(L * j, L)] for j in range(DV)]
                o = [o_rows[i, pl.ds(L * j, L)] for j in range(DV)]
                p = c[0] * o[0]
                for j in range(1, DV):
                    p = p + c[j] * o[j]
                pos_p[pl.ds(i * L, L)] = p
                for k in range(K):
                    r = i * K + k
                    q = c[0] * n_rows[r, pl.ds(0, L)]
                    for j in range(1, DV):
                        q = q + c[j] * n_rows[r, pl.ds(L * j, L)]
                    neg_p[pl.ds(r * L, L)] = q
                return carry

            lax.fori_loop(0, BC, b_body, 0)
            if g + 2 < NCHUNK:
                descs[par] = fire(g + 2, par)
            b0 = base + g * BC
            pltpu.sync_copy(pos_p, pos_h.at[pl.ds(b0 * L, BC * L)])
            pltpu.sync_copy(neg_p, negs_h.at[pl.ds(b0 * K * L, BC * K * L)])

    return sc_partials


def _loss_body(pos_ref, neg_ref, out_ref, *, inv_b):
    red = (jax.lax.broadcasted_iota(jnp.int32, (128, 8), 0) // L
           == jax.lax.broadcasted_iota(jnp.int32, (128, 8), 1)
           ).astype(jnp.float32)

    def log_sigmoid(x):
        return jnp.minimum(x, 0.0) - jnp.log1p(jnp.exp(-jnp.abs(x)))

    pos = jax.lax.dot(pos_ref[...], red,
                      preferred_element_type=jnp.float32)
    neg = jax.lax.dot(neg_ref[...], red,
                      preferred_element_type=jnp.float32)
    total = jnp.sum(log_sigmoid(pos)) + jnp.sum(log_sigmoid(-neg))
    out_ref[...] = jnp.full((1, 1), -total * inv_b, dtype=jnp.float32)


def kernel(center, context, negative, center_emb, context_emb):
    B, K = negative.shape
    V, D = center_emb.shape
    sc_partials = _make_sc_partials(B, K, D, V)
    pos_p, neg_p = sc_partials(
        center.astype(jnp.int32),
        context.astype(jnp.int32),
        negative.reshape(B * K).astype(jnp.int32),
        center_emb,
        context_emb,
    )
    loss = pl.pallas_call(
        functools.partial(_loss_body, inv_b=1.0 / B),
        out_shape=jax.ShapeDtypeStruct((1, 1), jnp.float32),
    )(pos_p.reshape(B * L // 128, 128), neg_p.reshape(B * K * L // 128, 128))
    return loss[0, 0]


# speedup vs baseline: 1.0809x; 1.0422x over previous
import functools

import jax
import jax.numpy as jnp
from jax import lax
from jax.experimental import pallas as pl
from jax.experimental.pallas import tpu as pltpu
from jax.experimental.pallas import tpu_sc as plsc

L = 16


@functools.lru_cache(maxsize=None)
def _make_sc_partials(B, K, D, V):
    info = plsc.get_sparse_core_info()
    NC, NS = info.num_cores, info.num_subcores
    NW = NC * NS
    assert B % NW == 0
    BPW = B // NW
    BC = 32
    assert BPW % BC == 0
    NCHUNK = BPW // BC
    DV = D // L

    mesh = plsc.VectorSubcoreMesh(core_axis_name="c", subcore_axis_name="s")

    row_buf = lambda n: pltpu.VMEM((n, D), jnp.float32)

    @functools.partial(
        pl.kernel,
        mesh=mesh,
        compiler_params=pltpu.CompilerParams(use_tc_tiling_on_sc=False),
        out_type=[
            jax.ShapeDtypeStruct((B * L,), jnp.float32),
            jax.ShapeDtypeStruct((B * K * L,), jnp.float32),
        ],
        scratch_types=[
            pltpu.VMEM((BPW,), jnp.int32),
            pltpu.VMEM((BPW,), jnp.int32),
            pltpu.VMEM((BPW * K,), jnp.int32),
            [row_buf(BC), row_buf(BC), row_buf(BC * K)],
            [row_buf(BC), row_buf(BC), row_buf(BC * K)],
            pltpu.VMEM((BC * L,), jnp.float32),
            pltpu.VMEM((BC * K * L,), jnp.float32),
            pltpu.SemaphoreType.DMA,
            pltpu.SemaphoreType.DMA,
        ],
    )
    def sc_partials(center_h, context_h, negflat_h, cemb_h, oemb_h,
                    pos_h, negs_h,
                    c_idx, o_idx, n_idx, bufs0, bufs1,
                    pos_p, neg_p, sem0, sem1):
        wid = lax.axis_index("s") * NC + lax.axis_index("c")
        base = pl.multiple_of(wid * BPW, BPW)
        pltpu.sync_copy(center_h.at[pl.ds(base, BPW)], c_idx)
        pltpu.sync_copy(context_h.at[pl.ds(base, BPW)], o_idx)
        pltpu.sync_copy(negflat_h.at[pl.ds(base * K, BPW * K)], n_idx)

        bufs = (bufs0, bufs1)
        sems = (sem0, sem1)

        def fire(g, par):
            c_rows, o_rows, n_rows = bufs[par]
            sem = sems[par]
            return (
                pltpu.async_copy(
                    cemb_h.at[c_idx.at[pl.ds(g * BC, BC)]], c_rows, sem),
                pltpu.async_copy(
                    oemb_h.at[o_idx.at[pl.ds(g * BC, BC)]], o_rows, sem),
                pltpu.async_copy(
                    oemb_h.at[n_idx.at[pl.ds(g * BC * K, BC * K)]], n_rows,
                    sem),
            )

        descs = {par: fire(par, par) for par in range(2)}
        for g in range(NCHUNK):
            par = g % 2
            for d in descs[par]:
                d.wait()
            c_rows, o_rows, n_rows = bufs[par]

            @plsc.parallel_loop(0, BC, unroll=2)
            def b_body(i, c_rows=c_rows, o_rows=o_rows, n_rows=n_rows):
                c = [c_rows[i, pl.ds(L * j, L)] for j in range(DV)]
                o = [o_rows[i, pl.ds(L * j, L)] for j in range(DV)]
                p = c[0] * o[0]
                for j in range(1, DV):
                    p = p + c[j] * o[j]
                pos_p[pl.ds(i * L, L)] = p
                for k in range(K):
                    r = i * K + k
                    q = c[0] * n_rows[r, pl.ds(0, L)]
                    for j in range(1, DV):
                        q = q + c[j] * n_rows[r, pl.ds(L * j, L)]
                    neg_p[pl.ds(r * L, L)] = q
            if g + 2 < NCHUNK:
                descs[par] = fire(g + 2, par)
            b0 = base + g * BC
            pltpu.sync_copy(pos_p, pos_h.at[pl.ds(b0 * L, BC * L)])
            pltpu.sync_copy(neg_p, negs_h.at[pl.ds(b0 * K * L, BC * K * L)])

    return sc_partials


def _loss_body(pos_ref, neg_ref, out_ref, *, inv_b):
    red = (jax.lax.broadcasted_iota(jnp.int32, (128, 8), 0) // L
           == jax.lax.broadcasted_iota(jnp.int32, (128, 8), 1)
           ).astype(jnp.float32)

    def log_sigmoid(x):
        return jnp.minimum(x, 0.0) - jnp.log1p(jnp.exp(-jnp.abs(x)))

    pos = jax.lax.dot(pos_ref[...], red,
                      preferred_element_type=jnp.float32)
    neg = jax.lax.dot(neg_ref[...], red,
                      preferred_element_type=jnp.float32)
    total = jnp.sum(log_sigmoid(pos)) + jnp.sum(log_sigmoid(-neg))
    out_ref[...] = jnp.full((1, 1), -total * inv_b, dtype=jnp.float32)


def kernel(center, context, negative, center_emb, context_emb):
    B, K = negative.shape
    V, D = center_emb.shape
    sc_partials = _make_sc_partials(B, K, D, V)
    pos_p, neg_p = sc_partials(
        center.astype(jnp.int32),
        context.astype(jnp.int32),
        negative.reshape(B * K).astype(jnp.int32),
        center_emb,
        context_emb,
    )
    loss = pl.pallas_call(
        functools.partial(_loss_body, inv_b=1.0 / B),
        out_shape=jax.ShapeDtypeStruct((1, 1), jnp.float32),
    )(pos_p.reshape(B * L // 128, 128), neg_p.reshape(B * K * L // 128, 128))
    return loss[0, 0]


# speedup vs baseline: 1.0871x; 1.0058x over previous
import functools

import jax
import jax.numpy as jnp
from jax import lax
from jax.experimental import pallas as pl
from jax.experimental.pallas import tpu as pltpu
from jax.experimental.pallas import tpu_sc as plsc

L = 16


@functools.lru_cache(maxsize=None)
def _make_sc_partials(B, K, D, V):
    info = plsc.get_sparse_core_info()
    NC, NS = info.num_cores, info.num_subcores
    NW = NC * NS
    assert B % NW == 0
    BPW = B // NW
    BC = 32
    assert BPW % BC == 0
    NCHUNK = BPW // BC
    DV = D // L

    mesh = plsc.VectorSubcoreMesh(core_axis_name="c", subcore_axis_name="s")

    row_buf = lambda n: pltpu.VMEM((n, D), jnp.float32)

    @functools.partial(
        pl.kernel,
        mesh=mesh,
        compiler_params=pltpu.CompilerParams(use_tc_tiling_on_sc=False),
        out_type=[
            jax.ShapeDtypeStruct((B * L,), jnp.float32),
            jax.ShapeDtypeStruct((B * K * L,), jnp.float32),
        ],
        scratch_types=[
            pltpu.VMEM((BPW,), jnp.int32),
            pltpu.VMEM((BPW,), jnp.int32),
            pltpu.VMEM((BPW * K,), jnp.int32),
            [row_buf(BC), row_buf(BC), row_buf(BC * K)],
            [row_buf(BC), row_buf(BC), row_buf(BC * K)],
            pltpu.VMEM((BC * L,), jnp.float32),
            pltpu.VMEM((BC * K * L,), jnp.float32),
            pltpu.SemaphoreType.DMA,
            pltpu.SemaphoreType.DMA,
        ],
    )
    def sc_partials(center_h, context_h, negflat_h, cemb_h, oemb_h,
                    pos_h, negs_h,
                    c_idx, o_idx, n_idx, bufs0, bufs1,
                    pos_p, neg_p, sem0, sem1):
        wid = lax.axis_index("s") * NC + lax.axis_index("c")
        base = pl.multiple_of(wid * BPW, BPW)
        pltpu.sync_copy(center_h.at[pl.ds(base, BPW)], c_idx)
        pltpu.sync_copy(context_h.at[pl.ds(base, BPW)], o_idx)
        pltpu.sync_copy(negflat_h.at[pl.ds(base * K, BPW * K)], n_idx)

        bufs = (bufs0, bufs1)
        sems = (sem0, sem1)

        def fire(g, par):
            c_rows, o_rows, n_rows = bufs[par]
            sem = sems[par]
            return (
                pltpu.async_copy(
                    cemb_h.at[c_idx.at[pl.ds(g * BC, BC)]], c_rows, sem),
                pltpu.async_copy(
                    oemb_h.at[o_idx.at[pl.ds(g * BC, BC)]], o_rows, sem),
                pltpu.async_copy(
                    oemb_h.at[n_idx.at[pl.ds(g * BC * K, BC * K)]], n_rows,
                    sem),
            )

        descs = {par: fire(par, par) for par in range(2)}
        for g in range(NCHUNK):
            par = g % 2
            for d in descs[par]:
                d.wait()
            c_rows, o_rows, n_rows = bufs[par]

            @plsc.parallel_loop(0, BC, unroll=4)
            def b_body(i, c_rows=c_rows, o_rows=o_rows, n_rows=n_rows):
                c = [c_rows[i, pl.ds(L * j, L)] for j in range(DV)]
                o = [o_rows[i, pl.ds(L * j, L)] for j in range(DV)]
                p = c[0] * o[0]
                for j in range(1, DV):
                    p = p + c[j] * o[j]
                pos_p[pl.ds(i * L, L)] = p
                for k in range(K):
                    r = i * K + k
                    q = c[0] * n_rows[r, pl.ds(0, L)]
                    for j in range(1, DV):
                        q = q + c[j] * n_rows[r, pl.ds(L * j, L)]
                    neg_p[pl.ds(r * L, L)] = q
            if g + 2 < NCHUNK:
                descs[par] = fire(g + 2, par)
            b0 = base + g * BC
            pltpu.sync_copy(pos_p, pos_h.at[pl.ds(b0 * L, BC * L)])
            pltpu.sync_copy(neg_p, negs_h.at[pl.ds(b0 * K * L, BC * K * L)])

    return sc_partials


def _loss_body(pos_ref, neg_ref, out_ref, *, inv_b):
    red = (jax.lax.broadcasted_iota(jnp.int32, (128, 8), 0) // L
           == jax.lax.broadcasted_iota(jnp.int32, (128, 8), 1)
           ).astype(jnp.float32)

    def log_sigmoid(x):
        return jnp.minimum(x, 0.0) - jnp.log1p(jnp.exp(-jnp.abs(x)))

    pos = jax.lax.dot(pos_ref[...], red,
                      preferred_element_type=jnp.float32)
    neg = jax.lax.dot(neg_ref[...], red,
                      preferred_element_type=jnp.float32)
    total = jnp.sum(log_sigmoid(pos)) + jnp.sum(log_sigmoid(-neg))
    out_ref[...] = jnp.full((1, 1), -total * inv_b, dtype=jnp.float32)


def kernel(center, context, negative, center_emb, context_emb):
    B, K = negative.shape
    V, D = center_emb.shape
    sc_partials = _make_sc_partials(B, K, D, V)
    pos_p, neg_p = sc_partials(
        center.astype(jnp.int32),
        context.astype(jnp.int32),
        negative.reshape(B * K).astype(jnp.int32),
        center_emb,
        context_emb,
    )
    loss = pl.pallas_call(
        functools.partial(_loss_body, inv_b=1.0 / B),
        out_shape=jax.ShapeDtypeStruct((1, 1), jnp.float32),
    )(pos_p.reshape(B * L // 128, 128), neg_p.reshape(B * K * L // 128, 128))
    return loss[0, 0]
